# baseline (speedup 1.0000x reference)
import functools

import jax
import jax.numpy as jnp
from jax import lax
from jax.experimental import pallas as pl
from jax.experimental.pallas import tpu as pltpu
from jax.experimental.pallas import tpu_sc as plsc

NC = 2
NS = 16
NW = NC * NS
CH = 128
G = 64
BR = 1000
SPLIT = (32, 128)


def _round_up(a, b):
    return (a + b - 1) // b * b


def _sc_counts(dst, z128, npad, epad):
    epw = epad // NW
    rps = npad // NS
    mesh = plsc.VectorSubcoreMesh(core_axis_name="c", subcore_axis_name="s")

    nch = epw // CH

    def k_body(dst_h, z128_h, cnt_h, dstm, onesv, cnts, sem):
        c = lax.axis_index("c")
        s = lax.axis_index("s")
        wid = c * NS + s
        r0 = s * rps
        pltpu.sync_copy(z128_h.at[pl.ds(r0, rps)], cnts.at[pl.ds(r0, rps)])
        pltpu.sync_copy(dst_h.at[pl.ds(wid * nch, nch)], dstm)

        @pl.loop(0, CH)
        def _(j):
            @pl.loop(0, 128, step=16)
            def _(l):
                onesv[j, pl.ds(l, 16)] = jnp.full((16,), 1.0, jnp.float32)

        plsc.subcore_barrier()

        @pl.loop(0, nch)
        def _(i):
            pltpu.sync_copy(onesv, cnts.at[dstm.at[i]], add=True)

        plsc.subcore_barrier()
        pltpu.sync_copy(cnts.at[pl.ds(r0, rps)], cnt_h.at[c].at[pl.ds(r0, rps)])

    kern = pl.kernel(
        k_body,
        out_type=jax.ShapeDtypeStruct((NC, npad, 128), jnp.float32),
        mesh=mesh,
        scratch_types=[
            pltpu.VMEM((nch, CH), jnp.int32),
            pltpu.VMEM((CH, 128), jnp.float32),
            pltpu.VMEM_SHARED((npad, 128), jnp.float32),
            pltpu.SemaphoreType.DMA,
        ],
    )
    return kern(dst.reshape(epad // CH, CH), z128)


def _sc_sums(table, src, dst, z128, npad, epad):
    epw = epad // NW
    rps = npad // NS
    mesh = plsc.VectorSubcoreMesh(core_axis_name="c", subcore_axis_name="s")

    nctot = epad // (NS * CH)
    if nctot == SPLIT[0] + SPLIT[1]:
        nch0, nch1 = SPLIT
    else:
        nch0 = nch1 = nctot // 2
    nhmax = max(nch0, nch1) // 2

    def k_body(table_h, src_h, dst_h, z128_h, sum_h,
               srcm, dstm, rows0, rows1, accs, sem0, sem1):
        rows = (rows0, rows1)
        sems = (sem0, sem1)
        c = lax.axis_index("c")
        s = lax.axis_index("s")

        def run_core(nch, base_chunk):
            nhalf = nch // 2
            ngrp = nhalf // 2
            for half in range(2):
                base = base_chunk + s * nch + half * nhalf
                pltpu.sync_copy(src_h.at[pl.ds(base, nhalf)],
                                srcm.at[pl.ds(0, nhalf)])
                pltpu.sync_copy(dst_h.at[pl.ds(base, nhalf)],
                                dstm.at[pl.ds(0, nhalf)])
                for b in range(2):
                    pltpu.async_copy(table_h.at[srcm.at[b]], rows[b], sems[b])

                @pl.loop(0, ngrp)
                def _(g):
                    ci = g * 2
                    for b in range(2):
                        pltpu.make_async_copy(table_h.at[srcm.at[ci + b]],
                                              rows[b], sems[b]).wait()
                        pltpu.sync_copy(rows[b], accs.at[dstm.at[ci + b]],
                                        add=True)

                        @pl.when(g < ngrp - 1)
                        def _():
                            pltpu.async_copy(table_h.at[srcm.at[ci + 2 + b]],
                                             rows[b], sems[b])

        rr = s * rps
        pltpu.sync_copy(z128_h.at[pl.ds(rr, rps)], accs.at[pl.ds(rr, rps)])
        plsc.subcore_barrier()

        @pl.when(c == 0)
        def _():
            run_core(nch0, 0)

        @pl.when(c == 1)
        def _():
            run_core(nch1, NS * nch0)

        plsc.subcore_barrier()
        pltpu.sync_copy(accs.at[pl.ds(rr, rps)], sum_h.at[c].at[pl.ds(rr, rps)])

    kern = pl.kernel(
        k_body,
        out_type=jax.ShapeDtypeStruct((NC, npad, 128), jnp.float32),
        mesh=mesh,
        scratch_types=[
            pltpu.VMEM((nhmax, CH), jnp.int32),
            pltpu.VMEM((nhmax, CH), jnp.int32),
            pltpu.VMEM((CH, 128), jnp.float32),
            pltpu.VMEM((CH, 128), jnp.float32),
            pltpu.VMEM_SHARED((npad, 128), jnp.float32),
            pltpu.SemaphoreType.DMA,
            pltpu.SemaphoreType.DMA,
        ],
    )
    return kern(table, src.reshape(epad // CH, CH),
                dst.reshape(epad // CH, CH), z128)


def _gelu_kernel(x_ref, o_ref):
    o_ref[...] = jax.nn.gelu(x_ref[...])


def _gelu_pallas(x):
    n, d = x.shape
    ng = n // BR
    return pl.pallas_call(
        _gelu_kernel,
        grid=(ng,),
        in_specs=[pl.BlockSpec((BR, d), lambda i: (i, 0))],
        out_specs=pl.BlockSpec((BR, d), lambda i: (i, 0)),
        out_shape=jax.ShapeDtypeStruct((n, d), jnp.float32),
    )(x)


def _x1_stats_kernel(sums_ref, cnts_ref, xg_ref, wl_ref, wr_ref, b_ref,
                     x1_ref, st_ref):
    i = pl.program_id(0)
    ssum = sums_ref[0] + sums_ref[1]
    cnt = cnts_ref[0, :, 0:1] + cnts_ref[1, :, 0:1]
    agg = ssum / jnp.maximum(cnt, 1.0)
    x1 = (jnp.dot(agg, wl_ref[...], preferred_element_type=jnp.float32)
          + jnp.dot(xg_ref[...], wr_ref[...], preferred_element_type=jnp.float32)
          + b_ref[...])
    x1_ref[...] = x1
    st = jnp.concatenate(
        [jnp.sum(x1, axis=0, keepdims=True),
         jnp.sum(x1 * x1, axis=0, keepdims=True)], axis=0)

    @pl.when(i == 0)
    def _():
        st_ref[...] = st

    @pl.when(i != 0)
    def _():
        st_ref[...] += st


def _x1_stats_pallas(sums, cnts, xg, wl, wr, b):
    n, d = xg.shape
    npad = sums.shape[1]
    ng = n // BR
    return pl.pallas_call(
        _x1_stats_kernel,
        grid=(ng,),
        in_specs=[
            pl.BlockSpec((NC, BR, 128), lambda i: (0, i, 0)),
            pl.BlockSpec((NC, BR, 128), lambda i: (0, i, 0)),
            pl.BlockSpec((BR, d), lambda i: (i, 0)),
            pl.BlockSpec((d, d), lambda i: (0, 0)),
            pl.BlockSpec((d, d), lambda i: (0, 0)),
            pl.BlockSpec((1, d), lambda i: (0, 0)),
        ],
        out_specs=[
            pl.BlockSpec((BR, d), lambda i: (i, 0)),
            pl.BlockSpec((2, d), lambda i: (0, 0)),
        ],
        out_shape=[
            jax.ShapeDtypeStruct((n, d), jnp.float32),
            jax.ShapeDtypeStruct((2, d), jnp.float32),
        ],
    )(sums, cnts, xg, wl, wr, b)


def _bn_gelu_kernel(nn, x1_ref, st_ref, g_ref, bt_ref, o_ref):
    mean = st_ref[0:1, :] / nn
    var = st_ref[1:2, :] / nn - mean * mean
    rstd = lax.rsqrt(var + 1e-5)
    o_ref[...] = jax.nn.gelu((x1_ref[...] - mean) * rstd * g_ref[...]
                             + bt_ref[...])


def _bn_gelu_pallas(x1, st, gamma, beta):
    n, d = x1.shape
    ng = n // BR
    return pl.pallas_call(
        functools.partial(_bn_gelu_kernel, float(n)),
        grid=(ng,),
        in_specs=[
            pl.BlockSpec((BR, d), lambda i: (i, 0)),
            pl.BlockSpec((2, d), lambda i: (0, 0)),
            pl.BlockSpec((1, d), lambda i: (0, 0)),
            pl.BlockSpec((1, d), lambda i: (0, 0)),
        ],
        out_specs=pl.BlockSpec((BR, d), lambda i: (i, 0)),
        out_shape=jax.ShapeDtypeStruct((n, d), jnp.float32),
    )(x1, st, gamma, beta)


def _xo_kernel(sums_ref, cnts_ref, h_ref, x1_ref, wl_ref, wr_ref, b_ref, o_ref):
    ssum = sums_ref[0] + sums_ref[1]
    cnt = cnts_ref[0, :, 0:1] + cnts_ref[1, :, 0:1]
    agg = ssum / jnp.maximum(cnt, 1.0)
    h2 = (jnp.dot(agg, wl_ref[...], preferred_element_type=jnp.float32)
          + jnp.dot(h_ref[...], wr_ref[...], preferred_element_type=jnp.float32)
          + b_ref[...])
    o_ref[...] = x1_ref[...] + h2


def _xo_pallas(sums, cnts, h, x1, wl, wr, b):
    n, d = h.shape
    ng = n // BR
    return pl.pallas_call(
        _xo_kernel,
        grid=(ng,),
        in_specs=[
            pl.BlockSpec((NC, BR, 128), lambda i: (0, i, 0)),
            pl.BlockSpec((NC, BR, 128), lambda i: (0, i, 0)),
            pl.BlockSpec((BR, d), lambda i: (i, 0)),
            pl.BlockSpec((BR, d), lambda i: (i, 0)),
            pl.BlockSpec((d, d), lambda i: (0, 0)),
            pl.BlockSpec((d, d), lambda i: (0, 0)),
            pl.BlockSpec((1, d), lambda i: (0, 0)),
        ],
        out_specs=pl.BlockSpec((BR, d), lambda i: (i, 0)),
        out_shape=jax.ShapeDtypeStruct((n, d), jnp.float32),
    )(sums, cnts, h, x1, wl, wr, b)


def _pool_kernel(ngrid, xo_ref, b_ref, wo_ref, bo_ref, o_ref,
                 acc_sum, acc_cnt, acc_min, acc_max):
    i = pl.program_id(0)

    @pl.when(i == 0)
    def _():
        acc_sum[...] = jnp.zeros_like(acc_sum)
        acc_cnt[...] = jnp.zeros_like(acc_cnt)
        acc_min[...] = jnp.full_like(acc_min, jnp.inf)
        acc_max[...] = jnp.full_like(acc_max, -jnp.inf)

    blk = xo_ref[...]
    bid = b_ref[...]
    g_lo = jnp.min(bid)
    g_hi = jnp.max(bid)
    giota = lax.broadcasted_iota(jnp.int32, (G, 1), 0)

    def body(g, carry):
        m = bid == g
        rowm = giota == g
        pmin = jnp.min(jnp.where(m, blk, jnp.inf), axis=0, keepdims=True)
        pmax = jnp.max(jnp.where(m, blk, -jnp.inf), axis=0, keepdims=True)
        psum = jnp.sum(jnp.where(m, blk, 0.0), axis=0, keepdims=True)
        pcnt = jnp.sum(m.astype(jnp.float32))
        acc_min[...] = jnp.where(rowm, jnp.minimum(acc_min[...], pmin), acc_min[...])
        acc_max[...] = jnp.where(rowm, jnp.maximum(acc_max[...], pmax), acc_max[...])
        acc_sum[...] = jnp.where(rowm, acc_sum[...] + psum, acc_sum[...])
        acc_cnt[...] = jnp.where(rowm, acc_cnt[...] + pcnt, acc_cnt[...])
        return carry

    lax.fori_loop(g_lo, g_hi + 1, body, 0)

    @pl.when(i == ngrid - 1)
    def _():
        mean = acc_sum[...] / jnp.maximum(acc_cnt[...], 1.0)
        pooled = jnp.concatenate([mean, acc_min[...], acc_max[...]], axis=1)
        o_ref[...] = (jnp.dot(pooled, wo_ref[...],
                              preferred_element_type=jnp.float32) + bo_ref[...])


def _pool_pallas(xo, batch2d, wo, bo):
    n, d = xo.shape
    ng = n // BR
    c = wo.shape[1]
    return pl.pallas_call(
        functools.partial(_pool_kernel, ng),
        grid=(ng,),
        in_specs=[
            pl.BlockSpec((BR, d), lambda i: (i, 0)),
            pl.BlockSpec((BR, 1), lambda i: (i, 0)),
            pl.BlockSpec((3 * d, c), lambda i: (0, 0)),
            pl.BlockSpec((1, c), lambda i: (0, 0)),
        ],
        out_specs=pl.BlockSpec((G, c), lambda i: (0, 0)),
        out_shape=jax.ShapeDtypeStruct((G, c), jnp.float32),
        scratch_shapes=[pltpu.VMEM((G, d), jnp.float32)] * 4,
    )(xo, batch2d, wo, bo)


def kernel(x, edge_index, batch, W1l, W1r, b1, gamma, beta, W2l, W2r, b2,
           Wout, bout):
    n, d = x.shape
    e = edge_index.shape[1]
    npad = _round_up(n + 1, NS * 8)
    epad = NS * CH * (SPLIT[0] + SPLIT[1])
    if epad < e:
        epad = _round_up(e, NW * 4 * CH)

    src = edge_index[0].astype(jnp.int32)
    dst = edge_index[1].astype(jnp.int32)
    if epad > e:
        src = jnp.concatenate([src, jnp.zeros((epad - e,), jnp.int32)])
        dst = jnp.concatenate([dst, jnp.full((epad - e,), npad - 1, jnp.int32)])

    z128 = jnp.zeros((npad, 128), jnp.float32)

    cnts1 = _sc_counts(dst, z128, npad, epad)
    xg = _gelu_pallas(x)
    sums1 = _sc_sums(xg, src, dst, z128, npad, epad)
    x1, st = _x1_stats_pallas(sums1, cnts1, xg, W1l.T, W1r.T,
                              b1.reshape(1, -1))
    h = _bn_gelu_pallas(x1, st, gamma.reshape(1, -1), beta.reshape(1, -1))
    sums2 = _sc_sums(h, src, dst, z128, npad, epad)
    xo = _xo_pallas(sums2, cnts1, h, x1, W2l.T, W2r.T, b2.reshape(1, -1))
    return _pool_pallas(xo, batch.reshape(-1, 1).astype(jnp.int32),
                        Wout.T, bout.reshape(1, -1))


# speedup vs baseline: 1.0967x; 1.0967x over previous
import functools

import jax
import jax.numpy as jnp
from jax import lax
from jax.experimental import pallas as pl
from jax.experimental.pallas import tpu as pltpu
from jax.experimental.pallas import tpu_sc as plsc

NC = 2
NS = 16
NW = NC * NS
CH = 128
G = 64
BR = 1000
SPLIT = (128, 32)


def _round_up(a, b):
    return (a + b - 1) // b * b


def _sc_counts(dst, z128, npad, epad):
    epw = epad // NW
    rps = npad // NS
    mesh = plsc.VectorSubcoreMesh(core_axis_name="c", subcore_axis_name="s")

    nch = epw // CH

    def k_body(dst_h, z128_h, cnt_h, dstm, onesv, cnts, sem):
        c = lax.axis_index("c")
        s = lax.axis_index("s")
        wid = c * NS + s
        r0 = s * rps
        pltpu.sync_copy(z128_h.at[pl.ds(r0, rps)], cnts.at[pl.ds(r0, rps)])
        pltpu.sync_copy(dst_h.at[pl.ds(wid * nch, nch)], dstm)

        @pl.loop(0, CH)
        def _(j):
            @pl.loop(0, 128, step=16)
            def _(l):
                onesv[j, pl.ds(l, 16)] = jnp.full((16,), 1.0, jnp.float32)

        plsc.subcore_barrier()

        @pl.loop(0, nch)
        def _(i):
            pltpu.sync_copy(onesv, cnts.at[dstm.at[i]], add=True)

        plsc.subcore_barrier()
        pltpu.sync_copy(cnts.at[pl.ds(r0, rps)], cnt_h.at[c].at[pl.ds(r0, rps)])

    kern = pl.kernel(
        k_body,
        out_type=jax.ShapeDtypeStruct((NC, npad, 128), jnp.float32),
        mesh=mesh,
        scratch_types=[
            pltpu.VMEM((nch, CH), jnp.int32),
            pltpu.VMEM((CH, 128), jnp.float32),
            pltpu.VMEM_SHARED((npad, 128), jnp.float32),
            pltpu.SemaphoreType.DMA,
        ],
    )
    return kern(dst.reshape(epad // CH, CH), z128)


def _sc_sums(table, src, dst, z128, npad, epad):
    epw = epad // NW
    rps = npad // NS
    mesh = plsc.VectorSubcoreMesh(core_axis_name="c", subcore_axis_name="s")

    nctot = epad // (NS * CH)
    if nctot == SPLIT[0] + SPLIT[1]:
        nch0, nch1 = SPLIT
    else:
        nch0 = nch1 = nctot // 2
    nhmax = max(nch0, nch1) // 2

    def k_body(table_h, src_h, dst_h, z128_h, sum_h,
               srcm, dstm, rows0, rows1, accs, sem0, sem1):
        rows = (rows0, rows1)
        sems = (sem0, sem1)
        c = lax.axis_index("c")
        s = lax.axis_index("s")

        def run_core(nch, base_chunk):
            nhalf = nch // 2
            ngrp = nhalf // 2
            for half in range(2):
                base = base_chunk + s * nch + half * nhalf
                pltpu.sync_copy(src_h.at[pl.ds(base, nhalf)],
                                srcm.at[pl.ds(0, nhalf)])
                pltpu.sync_copy(dst_h.at[pl.ds(base, nhalf)],
                                dstm.at[pl.ds(0, nhalf)])
                for b in range(2):
                    pltpu.async_copy(table_h.at[srcm.at[b]], rows[b], sems[b])

                @pl.loop(0, ngrp)
                def _(g):
                    ci = g * 2
                    for b in range(2):
                        pltpu.make_async_copy(table_h.at[srcm.at[ci + b]],
                                              rows[b], sems[b]).wait()
                        pltpu.sync_copy(rows[b], accs.at[dstm.at[ci + b]],
                                        add=True)

                        @pl.when(g < ngrp - 1)
                        def _():
                            pltpu.async_copy(table_h.at[srcm.at[ci + 2 + b]],
                                             rows[b], sems[b])

        rr = s * rps
        pltpu.sync_copy(z128_h.at[pl.ds(rr, rps)], accs.at[pl.ds(rr, rps)])
        plsc.subcore_barrier()

        @pl.when(c == 0)
        def _():
            run_core(nch0, 0)

        @pl.when(c == 1)
        def _():
            run_core(nch1, NS * nch0)

        plsc.subcore_barrier()
        pltpu.sync_copy(accs.at[pl.ds(rr, rps)], sum_h.at[c].at[pl.ds(rr, rps)])

    kern = pl.kernel(
        k_body,
        out_type=jax.ShapeDtypeStruct((NC, npad, 128), jnp.float32),
        mesh=mesh,
        scratch_types=[
            pltpu.VMEM((nhmax, CH), jnp.int32),
            pltpu.VMEM((nhmax, CH), jnp.int32),
            pltpu.VMEM((CH, 128), jnp.float32),
            pltpu.VMEM((CH, 128), jnp.float32),
            pltpu.VMEM_SHARED((npad, 128), jnp.float32),
            pltpu.SemaphoreType.DMA,
            pltpu.SemaphoreType.DMA,
        ],
    )
    return kern(table, src.reshape(epad // CH, CH),
                dst.reshape(epad // CH, CH), z128)


def _gelu_kernel(x_ref, o_ref):
    o_ref[...] = jax.nn.gelu(x_ref[...])


def _gelu_pallas(x):
    n, d = x.shape
    ng = n // BR
    return pl.pallas_call(
        _gelu_kernel,
        grid=(ng,),
        in_specs=[pl.BlockSpec((BR, d), lambda i: (i, 0))],
        out_specs=pl.BlockSpec((BR, d), lambda i: (i, 0)),
        out_shape=jax.ShapeDtypeStruct((n, d), jnp.float32),
    )(x)


def _x1_stats_kernel(sums_ref, cnts_ref, xg_ref, wl_ref, wr_ref, b_ref,
                     x1_ref, st_ref):
    i = pl.program_id(0)
    ssum = sums_ref[0] + sums_ref[1]
    cnt = cnts_ref[0, :, 0:1] + cnts_ref[1, :, 0:1]
    agg = ssum / jnp.maximum(cnt, 1.0)
    x1 = (jnp.dot(agg, wl_ref[...], preferred_element_type=jnp.float32)
          + jnp.dot(xg_ref[...], wr_ref[...], preferred_element_type=jnp.float32)
          + b_ref[...])
    x1_ref[...] = x1
    st = jnp.concatenate(
        [jnp.sum(x1, axis=0, keepdims=True),
         jnp.sum(x1 * x1, axis=0, keepdims=True)], axis=0)

    @pl.when(i == 0)
    def _():
        st_ref[...] = st

    @pl.when(i != 0)
    def _():
        st_ref[...] += st


def _x1_stats_pallas(sums, cnts, xg, wl, wr, b):
    n, d = xg.shape
    npad = sums.shape[1]
    ng = n // BR
    return pl.pallas_call(
        _x1_stats_kernel,
        grid=(ng,),
        in_specs=[
            pl.BlockSpec((NC, BR, 128), lambda i: (0, i, 0)),
            pl.BlockSpec((NC, BR, 128), lambda i: (0, i, 0)),
            pl.BlockSpec((BR, d), lambda i: (i, 0)),
            pl.BlockSpec((d, d), lambda i: (0, 0)),
            pl.BlockSpec((d, d), lambda i: (0, 0)),
            pl.BlockSpec((1, d), lambda i: (0, 0)),
        ],
        out_specs=[
            pl.BlockSpec((BR, d), lambda i: (i, 0)),
            pl.BlockSpec((2, d), lambda i: (0, 0)),
        ],
        out_shape=[
            jax.ShapeDtypeStruct((n, d), jnp.float32),
            jax.ShapeDtypeStruct((2, d), jnp.float32),
        ],
    )(sums, cnts, xg, wl, wr, b)


def _bn_gelu_kernel(nn, x1_ref, st_ref, g_ref, bt_ref, o_ref):
    mean = st_ref[0:1, :] / nn
    var = st_ref[1:2, :] / nn - mean * mean
    rstd = lax.rsqrt(var + 1e-5)
    o_ref[...] = jax.nn.gelu((x1_ref[...] - mean) * rstd * g_ref[...]
                             + bt_ref[...])


def _bn_gelu_pallas(x1, st, gamma, beta):
    n, d = x1.shape
    ng = n // BR
    return pl.pallas_call(
        functools.partial(_bn_gelu_kernel, float(n)),
        grid=(ng,),
        in_specs=[
            pl.BlockSpec((BR, d), lambda i: (i, 0)),
            pl.BlockSpec((2, d), lambda i: (0, 0)),
            pl.BlockSpec((1, d), lambda i: (0, 0)),
            pl.BlockSpec((1, d), lambda i: (0, 0)),
        ],
        out_specs=pl.BlockSpec((BR, d), lambda i: (i, 0)),
        out_shape=jax.ShapeDtypeStruct((n, d), jnp.float32),
    )(x1, st, gamma, beta)


def _xo_kernel(sums_ref, cnts_ref, h_ref, x1_ref, wl_ref, wr_ref, b_ref, o_ref):
    ssum = sums_ref[0] + sums_ref[1]
    cnt = cnts_ref[0, :, 0:1] + cnts_ref[1, :, 0:1]
    agg = ssum / jnp.maximum(cnt, 1.0)
    h2 = (jnp.dot(agg, wl_ref[...], preferred_element_type=jnp.float32)
          + jnp.dot(h_ref[...], wr_ref[...], preferred_element_type=jnp.float32)
          + b_ref[...])
    o_ref[...] = x1_ref[...] + h2


def _xo_pallas(sums, cnts, h, x1, wl, wr, b):
    n, d = h.shape
    ng = n // BR
    return pl.pallas_call(
        _xo_kernel,
        grid=(ng,),
        in_specs=[
            pl.BlockSpec((NC, BR, 128), lambda i: (0, i, 0)),
            pl.BlockSpec((NC, BR, 128), lambda i: (0, i, 0)),
            pl.BlockSpec((BR, d), lambda i: (i, 0)),
            pl.BlockSpec((BR, d), lambda i: (i, 0)),
            pl.BlockSpec((d, d), lambda i: (0, 0)),
            pl.BlockSpec((d, d), lambda i: (0, 0)),
            pl.BlockSpec((1, d), lambda i: (0, 0)),
        ],
        out_specs=pl.BlockSpec((BR, d), lambda i: (i, 0)),
        out_shape=jax.ShapeDtypeStruct((n, d), jnp.float32),
    )(sums, cnts, h, x1, wl, wr, b)


def _pool_kernel(ngrid, xo_ref, b_ref, wo_ref, bo_ref, o_ref,
                 acc_sum, acc_cnt, acc_min, acc_max):
    i = pl.program_id(0)

    @pl.when(i == 0)
    def _():
        acc_sum[...] = jnp.zeros_like(acc_sum)
        acc_cnt[...] = jnp.zeros_like(acc_cnt)
        acc_min[...] = jnp.full_like(acc_min, jnp.inf)
        acc_max[...] = jnp.full_like(acc_max, -jnp.inf)

    blk = xo_ref[...]
    bid = b_ref[...]
    g_lo = jnp.min(bid)
    g_hi = jnp.max(bid)
    giota = lax.broadcasted_iota(jnp.int32, (G, 1), 0)

    def body(g, carry):
        m = bid == g
        rowm = giota == g
        pmin = jnp.min(jnp.where(m, blk, jnp.inf), axis=0, keepdims=True)
        pmax = jnp.max(jnp.where(m, blk, -jnp.inf), axis=0, keepdims=True)
        psum = jnp.sum(jnp.where(m, blk, 0.0), axis=0, keepdims=True)
        pcnt = jnp.sum(m.astype(jnp.float32))
        acc_min[...] = jnp.where(rowm, jnp.minimum(acc_min[...], pmin), acc_min[...])
        acc_max[...] = jnp.where(rowm, jnp.maximum(acc_max[...], pmax), acc_max[...])
        acc_sum[...] = jnp.where(rowm, acc_sum[...] + psum, acc_sum[...])
        acc_cnt[...] = jnp.where(rowm, acc_cnt[...] + pcnt, acc_cnt[...])
        return carry

    lax.fori_loop(g_lo, g_hi + 1, body, 0)

    @pl.when(i == ngrid - 1)
    def _():
        mean = acc_sum[...] / jnp.maximum(acc_cnt[...], 1.0)
        pooled = jnp.concatenate([mean, acc_min[...], acc_max[...]], axis=1)
        o_ref[...] = (jnp.dot(pooled, wo_ref[...],
                              preferred_element_type=jnp.float32) + bo_ref[...])


def _pool_pallas(xo, batch2d, wo, bo):
    n, d = xo.shape
    ng = n // BR
    c = wo.shape[1]
    return pl.pallas_call(
        functools.partial(_pool_kernel, ng),
        grid=(ng,),
        in_specs=[
            pl.BlockSpec((BR, d), lambda i: (i, 0)),
            pl.BlockSpec((BR, 1), lambda i: (i, 0)),
            pl.BlockSpec((3 * d, c), lambda i: (0, 0)),
            pl.BlockSpec((1, c), lambda i: (0, 0)),
        ],
        out_specs=pl.BlockSpec((G, c), lambda i: (0, 0)),
        out_shape=jax.ShapeDtypeStruct((G, c), jnp.float32),
        scratch_shapes=[pltpu.VMEM((G, d), jnp.float32)] * 4,
    )(xo, batch2d, wo, bo)


def kernel(x, edge_index, batch, W1l, W1r, b1, gamma, beta, W2l, W2r, b2,
           Wout, bout):
    n, d = x.shape
    e = edge_index.shape[1]
    npad = _round_up(n + 1, NS * 8)
    epad = NS * CH * (SPLIT[0] + SPLIT[1])
    if epad < e:
        epad = _round_up(e, NW * 4 * CH)

    src = edge_index[0].astype(jnp.int32)
    dst = edge_index[1].astype(jnp.int32)
    if epad > e:
        src = jnp.concatenate([src, jnp.zeros((epad - e,), jnp.int32)])
        dst = jnp.concatenate([dst, jnp.full((epad - e,), npad - 1, jnp.int32)])

    z128 = jnp.zeros((npad, 128), jnp.float32)

    cnts1 = _sc_counts(dst, z128, npad, epad)
    xg = _gelu_pallas(x)
    sums1 = _sc_sums(xg, src, dst, z128, npad, epad)
    x1, st = _x1_stats_pallas(sums1, cnts1, xg, W1l.T, W1r.T,
                              b1.reshape(1, -1))
    h = _bn_gelu_pallas(x1, st, gamma.reshape(1, -1), beta.reshape(1, -1))
    sums2 = _sc_sums(h, src, dst, z128, npad, epad)
    xo = _xo_pallas(sums2, cnts1, h, x1, W2l.T, W2r.T, b2.reshape(1, -1))
    return _pool_pallas(xo, batch.reshape(-1, 1).astype(jnp.int32),
                        Wout.T, bout.reshape(1, -1))
